# Initial kernel scaffold; baseline (speedup 1.0000x reference)
import jax
import jax.numpy as jnp
from jax.experimental import pallas as pl


def kernel(Pred, targets):
    raise NotImplementedError("write your pallas kernel here")


# speedup vs baseline: 145.2090x; 145.2090x over previous
import jax
import jax.numpy as jnp
from jax.experimental import pallas as pl
from jax.experimental.pallas import tpu as pltpu

_DELTA = 0.1
_INV2D = 1.0 / (2.0 * _DELTA)


def _prep_kernel(l_ref, t_ref, fg_ref, bg_ref, stats_ref):
    l = l_ref[...]
    t = t_ref[...]
    fg = t > 0.0
    fgf = jnp.where(fg, 1.0, 0.0).astype(jnp.float32)
    fg_or_inf = jnp.where(fg, l, jnp.inf)
    thr = jnp.min(fg_or_inf) - _DELTA
    bgf = jnp.where(jnp.logical_not(fg) & (l >= thr), 1.0, 0.0).astype(jnp.float32)
    fg_ref[...] = fgf
    bg_ref[...] = bgf
    stats_ref[...] = jnp.full((1, 1), jnp.sum(fgf), dtype=jnp.float32)


def _pair_kernel(xcol_ref, tcol_ref, lrow_ref, fg_ref, bg_ref,
                 out_ref, accfp_ref, accrp_ref):
    j = pl.program_id(1)

    @pl.when(j == 0)
    def _():
        accfp_ref[...] = jnp.zeros_like(accfp_ref)
        accrp_ref[...] = jnp.zeros_like(accrp_ref)

    x = xcol_ref[...]
    l = lrow_ref[...]
    rel = jnp.clip((l - x) * _INV2D + 0.5, 0.0, 1.0)
    accfp_ref[...] += jnp.sum(bg_ref[...] * rel, axis=1, keepdims=True)
    accrp_ref[...] += jnp.sum(fg_ref[...] * rel, axis=1, keepdims=True)

    @pl.when(j == pl.num_programs(1) - 1)
    def _():
        i = pl.program_id(0)

        @pl.when(i == 0)
        def _():
            out_ref[...] = jnp.zeros_like(out_ref)

        fp = accfp_ref[...]
        rank = fp + accrp_ref[...]
        fgx = jnp.where(tcol_ref[...] > 0.0, 1.0, 0.0).astype(jnp.float32)
        err = fgx * fp / jnp.where(rank > 0.0, rank, 1.0)
        out_ref[...] += jnp.full((1, 1), jnp.sum(err), dtype=jnp.float32)


def kernel(Pred, targets):
    l_flat = Pred.reshape(-1).astype(jnp.float32)
    t_flat = targets.reshape(-1).astype(jnp.float32)
    N = l_flat.shape[0]
    l_row = l_flat.reshape(1, N)
    t_row = t_flat.reshape(1, N)

    fg_row, bg_row, fg_num = pl.pallas_call(
        _prep_kernel,
        out_shape=[
            jax.ShapeDtypeStruct((1, N), jnp.float32),
            jax.ShapeDtypeStruct((1, N), jnp.float32),
            jax.ShapeDtypeStruct((1, 1), jnp.float32),
        ],
    )(l_row, t_row)

    BX = 256
    BL = 2048
    grid = (N // BX, N // BL)

    err_sum = pl.pallas_call(
        _pair_kernel,
        grid=grid,
        in_specs=[
            pl.BlockSpec((BX, 1), lambda i, j: (i, 0)),
            pl.BlockSpec((BX, 1), lambda i, j: (i, 0)),
            pl.BlockSpec((1, BL), lambda i, j: (0, j)),
            pl.BlockSpec((1, BL), lambda i, j: (0, j)),
            pl.BlockSpec((1, BL), lambda i, j: (0, j)),
        ],
        out_specs=pl.BlockSpec((1, 1), lambda i, j: (0, 0)),
        out_shape=jax.ShapeDtypeStruct((1, 1), jnp.float32),
        scratch_shapes=[
            pltpu.VMEM((BX, 1), jnp.float32),
            pltpu.VMEM((BX, 1), jnp.float32),
        ],
    )(l_flat.reshape(N, 1), t_flat.reshape(N, 1), l_row, fg_row, bg_row)

    fg_n = fg_num[0, 0]
    s = err_sum[0, 0]
    return jnp.where(fg_n > 0.0, s / jnp.where(fg_n > 0.0, fg_n, 1.0), 0.0)


# speedup vs baseline: 199.7151x; 1.3754x over previous
import functools

import jax
import jax.numpy as jnp
from jax import lax
from jax.experimental import pallas as pl
from jax.experimental.pallas import tpu as pltpu
from jax.experimental.pallas import tpu_sc as plsc

_DELTA = 0.1
_INV2D = 1.0 / (2.0 * _DELTA)


def _sc_compact_body(l_hbm, t_hbm, fgx_hbm, fgl_hbm, bgl_hbm, cnt_hbm,
                     l_v, t_v, fgx_v, fgl_v, bgl_v, cnt_v):
    N = l_v.shape[0]
    cid = lax.axis_index("c")
    sid = lax.axis_index("s")

    @pl.when((cid == 0) & (sid == 0))
    def _():
        pltpu.sync_copy(l_hbm, l_v)
        pltpu.sync_copy(t_hbm, t_v)

        def fill(i, carry):
            off = pl.ds(i * 16, 16)
            fgx_v[off] = jnp.full((16,), jnp.inf, jnp.float32)
            fgl_v[off] = jnp.full((16,), -jnp.inf, jnp.float32)
            bgl_v[off] = jnp.full((16,), -jnp.inf, jnp.float32)
            return carry

        lax.fori_loop(0, (N + 16) // 16, fill, 0)

        def step(i, carry):
            nfg, nbg = carry
            lv = l_v[pl.ds(i * 16, 16)]
            tv = t_v[pl.ds(i * 16, 16)]
            m = tv > 0.0
            nm = jnp.logical_not(m)
            mi = m.astype(jnp.int32)
            csf = plsc.cumsum(mi)
            posf = nfg + csf - 1
            plsc.store_scatter(fgx_v, [posf], lv, mask=m)
            plsc.store_scatter(fgl_v, [posf], lv, mask=m)
            csb = plsc.cumsum(1 - mi)
            posb = nbg + csb - 1
            plsc.store_scatter(bgl_v, [posb], lv, mask=nm)
            c = jnp.max(csf)
            return nfg + c, nbg + (16 - c)

        nfg, nbg = lax.fori_loop(0, N // 16, step,
                                 (jnp.int32(0), jnp.int32(0)))

        iota = lax.iota(jnp.int32, 16)
        nfg_v = jnp.broadcast_to(nfg, (16,))
        nbg_v = jnp.broadcast_to(nbg, (16,))
        cnt_v[...] = jnp.where(iota == 0, nfg_v, jnp.where(iota == 1, nbg_v, 0))
        pltpu.sync_copy(fgx_v.at[pl.ds(0, N)], fgx_hbm)
        pltpu.sync_copy(fgl_v.at[pl.ds(0, N)], fgl_hbm)
        pltpu.sync_copy(bgl_v.at[pl.ds(0, N)], bgl_hbm)
        pltpu.sync_copy(cnt_v, cnt_hbm)


def _sc_compact(l_flat, t_flat):
    N = l_flat.shape[0]
    mesh = plsc.VectorSubcoreMesh(core_axis_name="c", subcore_axis_name="s")
    f = pl.kernel(
        _sc_compact_body,
        out_type=[
            jax.ShapeDtypeStruct((N,), jnp.float32),
            jax.ShapeDtypeStruct((N,), jnp.float32),
            jax.ShapeDtypeStruct((N,), jnp.float32),
            jax.ShapeDtypeStruct((16,), jnp.int32),
        ],
        mesh=mesh,
        compiler_params=pltpu.CompilerParams(needs_layout_passes=False),
        scratch_types=[
            pltpu.VMEM((N,), jnp.float32),
            pltpu.VMEM((N,), jnp.float32),
            pltpu.VMEM((N + 16,), jnp.float32),
            pltpu.VMEM((N + 16,), jnp.float32),
            pltpu.VMEM((N + 16,), jnp.float32),
            pltpu.VMEM((16,), jnp.int32),
        ],
    )
    return f(l_flat, t_flat)


def _pair_kernel(cnt_ref, xcol_ref, fgl_ref, bgl_ref,
                 out_ref, accfp_ref, accrp_ref):
    i = pl.program_id(0)
    j = pl.program_id(1)
    BX = xcol_ref.shape[0]
    BL = fgl_ref.shape[1]
    nfg = cnt_ref[0]
    nbg = cnt_ref[1]

    @pl.when((i == 0) & (j == 0))
    def _():
        out_ref[...] = jnp.zeros_like(out_ref)

    @pl.when(i * BX < nfg)
    def _():
        @pl.when(j == 0)
        def _():
            accfp_ref[...] = jnp.zeros_like(accfp_ref)
            accrp_ref[...] = jnp.zeros_like(accrp_ref)

        x = xcol_ref[...] * _INV2D

        @pl.when(j * BL < nfg)
        def _():
            relf = jnp.clip(fgl_ref[...] - x, 0.0, 1.0)
            accrp_ref[...] += jnp.sum(relf, axis=1, keepdims=True)

        @pl.when(j * BL < nbg)
        def _():
            relb = jnp.clip(bgl_ref[...] - x, 0.0, 1.0)
            accfp_ref[...] += jnp.sum(relb, axis=1, keepdims=True)

        @pl.when(j == pl.num_programs(1) - 1)
        def _():
            fp = accfp_ref[...]
            rank = fp + accrp_ref[...]
            err = fp / jnp.where(rank > 0.0, rank, 1.0)
            out_ref[...] += jnp.full((1, 1), jnp.sum(err), dtype=jnp.float32)


def _pair_stage(fgx, fgl, bgl, cnt):
    N = fgx.shape[0]
    BX = 256
    BL = 2048
    fgl_row = (fgl * _INV2D + 0.5).reshape(1, N)
    bgl_row = (bgl * _INV2D + 0.5).reshape(1, N)
    return pl.pallas_call(
        _pair_kernel,
        grid=(N // BX, N // BL),
        in_specs=[
            pl.BlockSpec(memory_space=pltpu.SMEM),
            pl.BlockSpec((BX, 1), lambda i, j: (i, 0)),
            pl.BlockSpec((1, BL), lambda i, j: (0, j)),
            pl.BlockSpec((1, BL), lambda i, j: (0, j)),
        ],
        out_specs=pl.BlockSpec((1, 1), lambda i, j: (0, 0)),
        out_shape=jax.ShapeDtypeStruct((1, 1), jnp.float32),
        scratch_shapes=[
            pltpu.VMEM((BX, 1), jnp.float32),
            pltpu.VMEM((BX, 1), jnp.float32),
        ],
    )(cnt, fgx.reshape(N, 1), fgl_row, bgl_row)


def kernel(Pred, targets):
    l_flat = Pred.reshape(-1).astype(jnp.float32)
    t_flat = targets.reshape(-1).astype(jnp.float32)
    fgx, fgl, bgl, cnt = _sc_compact(l_flat, t_flat)
    err_sum = _pair_stage(fgx, fgl, bgl, cnt)
    fg_n = cnt[0].astype(jnp.float32)
    s = err_sum[0, 0]
    return jnp.where(fg_n > 0.0, s / jnp.where(fg_n > 0.0, fg_n, 1.0), 0.0)


# speedup vs baseline: 469.0286x; 2.3485x over previous
import functools

import jax
import jax.numpy as jnp
from jax import lax
from jax.experimental import pallas as pl
from jax.experimental.pallas import tpu as pltpu
from jax.experimental.pallas import tpu_sc as plsc

_DELTA = 0.1
_INV2D = 1.0 / (2.0 * _DELTA)


def _sc_compact_body(l_hbm, t_hbm, fgx_hbm, fgl_hbm, bgl_hbm, cnt_hbm,
                     l_v, t_v, fgx_v, fgl_v, bgl_v, cnt_v):
    N = l_v.shape[0]
    cid = lax.axis_index("c")
    sid = lax.axis_index("s")

    @pl.when((cid == 0) & (sid == 0))
    def _():
        pltpu.sync_copy(l_hbm, l_v)
        pltpu.sync_copy(t_hbm, t_v)

        def fill(i, carry):
            off = pl.ds(i * 16, 16)
            fgx_v[off] = jnp.full((16,), jnp.inf, jnp.float32)
            fgl_v[off] = jnp.full((16,), -jnp.inf, jnp.float32)
            bgl_v[off] = jnp.full((16,), -jnp.inf, jnp.float32)
            return carry

        lax.fori_loop(0, (N + 16) // 16, fill, 0)

        def step(i, carry):
            nfg, nbg = carry
            lv = l_v[pl.ds(i * 16, 16)]
            tv = t_v[pl.ds(i * 16, 16)]
            m = tv > 0.0
            nm = jnp.logical_not(m)
            mi = m.astype(jnp.int32)
            csf = plsc.cumsum(mi)
            posf = nfg + csf - 1
            plsc.store_scatter(fgx_v, [posf], lv, mask=m)
            plsc.store_scatter(fgl_v, [posf], lv, mask=m)
            csb = plsc.cumsum(1 - mi)
            posb = nbg + csb - 1
            plsc.store_scatter(bgl_v, [posb], lv, mask=nm)
            c = jnp.max(csf)
            return nfg + c, nbg + (16 - c)

        nfg, nbg = lax.fori_loop(0, N // 16, step,
                                 (jnp.int32(0), jnp.int32(0)))

        iota = lax.iota(jnp.int32, 16)
        nfg_v = jnp.broadcast_to(nfg, (16,))
        nbg_v = jnp.broadcast_to(nbg, (16,))
        cnt_v[...] = jnp.where(iota == 0, nfg_v, jnp.where(iota == 1, nbg_v, 0))
        pltpu.sync_copy(fgx_v.at[pl.ds(0, N)], fgx_hbm)
        pltpu.sync_copy(fgl_v.at[pl.ds(0, N)], fgl_hbm)
        pltpu.sync_copy(bgl_v.at[pl.ds(0, N)], bgl_hbm)
        pltpu.sync_copy(cnt_v, cnt_hbm)


def _sc_compact(l_flat, t_flat):
    N = l_flat.shape[0]
    mesh = plsc.VectorSubcoreMesh(core_axis_name="c", subcore_axis_name="s")
    f = pl.kernel(
        _sc_compact_body,
        out_type=[
            jax.ShapeDtypeStruct((N,), jnp.float32),
            jax.ShapeDtypeStruct((N,), jnp.float32),
            jax.ShapeDtypeStruct((N,), jnp.float32),
            jax.ShapeDtypeStruct((16,), jnp.int32),
        ],
        mesh=mesh,
        compiler_params=pltpu.CompilerParams(needs_layout_passes=False),
        scratch_types=[
            pltpu.VMEM((N,), jnp.float32),
            pltpu.VMEM((N,), jnp.float32),
            pltpu.VMEM((N + 16,), jnp.float32),
            pltpu.VMEM((N + 16,), jnp.float32),
            pltpu.VMEM((N + 16,), jnp.float32),
            pltpu.VMEM((16,), jnp.int32),
        ],
    )
    return f(l_flat, t_flat)


def _pair_kernel(cnt_ref, xcol_ref, fgl_ref, bgl_ref,
                 out_ref, accfp_ref, accrp_ref):
    i = pl.program_id(0)
    BX = xcol_ref.shape[0]
    NJ, _, BL = fgl_ref.shape
    NT = BL // 128
    nfg = cnt_ref[0]
    nbg = cnt_ref[1]

    @pl.when(i == 0)
    def _():
        out_ref[...] = jnp.zeros_like(out_ref)

    @pl.when(i * BX < nfg)
    def _():
        x = xcol_ref[...] * _INV2D
        nj_fg = lax.div(nfg + BL - 1, BL)
        nj_bg = lax.div(nbg + BL - 1, BL)
        accfp_ref[...] = jnp.zeros_like(accfp_ref)
        accrp_ref[...] = jnp.zeros_like(accrp_ref)

        def make_step(l3_ref, acc_ref):
            def step(jj, carry):
                rel = jnp.clip(l3_ref[jj] - x, 0.0, 1.0)
                part = rel[:, 0:128]
                for t in range(1, NT):
                    part = part + rel[:, t * 128:(t + 1) * 128]
                acc_ref[...] += part
                return carry
            return step

        lax.fori_loop(0, nj_fg, make_step(fgl_ref, accrp_ref), 0)
        lax.fori_loop(0, nj_bg, make_step(bgl_ref, accfp_ref), 0)

        fp = jnp.sum(accfp_ref[...], axis=1, keepdims=True)
        rp = jnp.sum(accrp_ref[...], axis=1, keepdims=True)
        rank = fp + rp
        err = fp / jnp.where(rank > 0.0, rank, 1.0)
        out_ref[...] += jnp.full((1, 1), jnp.sum(err), dtype=jnp.float32)


def _pair_stage(fgx, fgl, bgl, cnt):
    N = fgx.shape[0]
    BX = 512
    BL = 2048
    NJ = N // BL
    fgl_3d = (fgl * _INV2D + 0.5).reshape(NJ, 1, BL)
    bgl_3d = (bgl * _INV2D + 0.5).reshape(NJ, 1, BL)
    return pl.pallas_call(
        _pair_kernel,
        grid=(N // BX,),
        in_specs=[
            pl.BlockSpec(memory_space=pltpu.SMEM),
            pl.BlockSpec((BX, 1), lambda i: (i, 0)),
            pl.BlockSpec((NJ, 1, BL), lambda i: (0, 0, 0)),
            pl.BlockSpec((NJ, 1, BL), lambda i: (0, 0, 0)),
        ],
        out_specs=pl.BlockSpec((1, 1), lambda i: (0, 0)),
        out_shape=jax.ShapeDtypeStruct((1, 1), jnp.float32),
        scratch_shapes=[
            pltpu.VMEM((BX, 128), jnp.float32),
            pltpu.VMEM((BX, 128), jnp.float32),
        ],
    )(cnt, fgx.reshape(N, 1), fgl_3d, bgl_3d)


def kernel(Pred, targets):
    l_flat = Pred.reshape(-1).astype(jnp.float32)
    t_flat = targets.reshape(-1).astype(jnp.float32)
    fgx, fgl, bgl, cnt = _sc_compact(l_flat, t_flat)
    err_sum = _pair_stage(fgx, fgl, bgl, cnt)
    fg_n = cnt[0].astype(jnp.float32)
    s = err_sum[0, 0]
    return jnp.where(fg_n > 0.0, s / jnp.where(fg_n > 0.0, fg_n, 1.0), 0.0)
